# baseline (speedup 1.0000x reference)
import functools

import jax
import jax.numpy as jnp
from jax import lax
from jax.experimental import pallas as pl
from jax.experimental.pallas import tpu as pltpu
from jax.experimental.pallas import tpu_sc as plsc

B, R, S = 256, 64, 64
N_PTS = 20000
K = 3
PAD_D = 16
N_IDX = 2 * B * R
N_WORKERS = 32
IDX_PER_W = N_IDX // N_WORKERS


PER_W = B * R // N_WORKERS


def _sc_gather_body(ref_flat, src_flat, refi_hbm, srci_hbm, out_hbm,
                    refi_full, srci_full, idx3_v, vals_v, sem):
    wid = lax.axis_index("s") * 2 + lax.axis_index("c")
    base = wid * PER_W
    cp_r = pltpu.async_copy(refi_hbm, refi_full, sem)
    cp_s = pltpu.async_copy(srci_hbm, srci_full, sem)
    cp_r.wait()
    cp_s.wait()
    copies = []
    for cloud, full in enumerate((refi_full, srci_full)):
        for c in range(3):
            k = 3 * cloud + c
            for j in range(PER_W // 16):
                half = j // (PER_W // 32)
                bvec = lax.iota(jnp.int32, 16) + (j * 16 - half * B)
                rvec = lax.iota(jnp.int32, 16) * 0 + (2 * wid + half)
                iv = plsc.load_gather(full, [bvec, rvec])
                idx3_v[k][pl.ds(j * 16, 16)] = iv * 3 + c
            copies.append(pltpu.async_copy(ref_flat.at[idx3_v[k]] if cloud == 0
                                           else src_flat.at[idx3_v[k]],
                                           vals_v[k], sem))
    for k, cp in enumerate(copies):
        cp.wait()
        pltpu.sync_copy(vals_v[k], out_hbm.at[k, pl.ds(base, PER_W)])


@functools.cache
def _sc_gather():
    return pl.kernel(
        _sc_gather_body,
        out_type=jax.ShapeDtypeStruct((6, B * R), jnp.float32),
        mesh=plsc.VectorSubcoreMesh(core_axis_name="c", subcore_axis_name="s"),
        scratch_types=[
            pltpu.VMEM((B, R), jnp.int32),
            pltpu.VMEM((B, R), jnp.int32),
            [pltpu.VMEM((PER_W,), jnp.int32) for _ in range(6)],
            [pltpu.VMEM((PER_W,), jnp.float32) for _ in range(6)],
            pltpu.SemaphoreType.DMA,
        ],
        compiler_params=pltpu.CompilerParams(use_tc_tiling_on_sc=False,
                                             needs_layout_passes=False),
    )


def _jacobi4(n_mat, v_mat, sweeps=6):
    def get(i, j):
        return n_mat[(i, j)] if i <= j else n_mat[(j, i)]

    def put(i, j, val):
        n_mat[(i, j) if i <= j else (j, i)] = val

    for _ in range(sweeps):
        for p in range(4):
            for q in range(p + 1, 4):
                apq = get(p, q)
                app = get(p, p)
                aqq = get(q, q)
                tau = (aqq - app) / (2.0 * apq + 1e-30)
                t = jnp.sign(tau) / (jnp.abs(tau) + jnp.sqrt(1.0 + tau * tau))
                small = jnp.abs(apq) < 1e-12
                c = jnp.where(small, 1.0, 1.0 / jnp.sqrt(1.0 + t * t))
                s = jnp.where(small, 0.0, t * c)
                for k in range(4):
                    if k != p and k != q:
                        akp = get(k, p)
                        akq = get(k, q)
                        put(k, p, c * akp - s * akq)
                        put(k, q, s * akp + c * akq)
                put(p, p, app - t * apq)
                put(q, q, aqq + t * apq)
                put(p, q, jnp.float32(0.0) * apq)
                for k in range(4):
                    vkp = v_mat[(k, p)]
                    vkq = v_mat[(k, q)]
                    v_mat[(k, p)] = c * vkp - s * vkq
                    v_mat[(k, q)] = s * vkp + c * vkq
    return [n_mat[(i, i)] for i in range(4)], v_mat


def _tc_body(score_ref, planes_ref, maskr_ref, masks_ref, conf_ref, out_ref):
    f32 = jnp.float32
    e = score_ref[...]
    conf = conf_ref[0, 0]
    logconf = jnp.log(conf)
    neg = f32(-jnp.inf)

    m1 = jnp.max(e, axis=1, keepdims=True)
    e1 = jnp.where(e >= m1, neg, e)
    m2 = jnp.max(e1, axis=1, keepdims=True)
    e2 = jnp.where(e1 >= m2, neg, e1)
    m3r = jnp.max(e2, axis=1, keepdims=True)

    c1 = jnp.max(e, axis=0, keepdims=True)
    f1 = jnp.where(e >= c1, neg, e)
    c2 = jnp.max(f1, axis=0, keepdims=True)
    f2 = jnp.where(f1 >= c2, neg, f1)
    m3c = jnp.max(f2, axis=0, keepdims=True)

    maskr = maskr_ref[...].astype(f32)
    masks = masks_ref[...].astype(f32)
    w = jnp.where((e >= m3r) & (e >= m3c) & (e > logconf), f32(1.0), f32(0.0))
    w = w * maskr[:, None, :] * masks[None, :, :]

    wr = jnp.sum(w, axis=1)
    ws = jnp.sum(w, axis=0)
    w_total = jnp.sum(wr)

    refc_p = [planes_ref[i] for i in range(3)]
    srcc_p = [planes_ref[3 + i] for i in range(3)]

    p_s = [jnp.sum(wr * rp) for rp in refc_p]
    q_s = [jnp.sum(ws * sp) for sp in srcc_p]

    g_s = []
    for d in range(3):
        yd = jnp.sum(w * srcc_p[d][None, :, :], axis=1)
        g_s.append([jnp.sum(yd * rp) for rp in refc_p])

    sw = w_total + f32(1e-8)
    ref_c = [p / sw for p in p_s]
    src_c = [q / sw for q in q_s]
    h = [[g_s[d][ee] - src_c[d] * p_s[ee] - q_s[d] * ref_c[ee]
          + w_total * src_c[d] * ref_c[ee] for ee in range(3)] for d in range(3)]

    sxx, sxy, sxz = h[0]
    syx, syy, syz = h[1]
    szx, szy, szz = h[2]
    n_mat = {
        (0, 0): sxx + syy + szz, (0, 1): syz - szy, (0, 2): szx - sxz, (0, 3): sxy - syx,
        (1, 1): sxx - syy - szz, (1, 2): sxy + syx, (1, 3): szx + sxz,
        (2, 2): -sxx + syy - szz, (2, 3): syz + szy,
        (3, 3): -sxx - syy + szz,
    }
    v_mat = {(i, j): f32(1.0) if i == j else f32(0.0)
             for i in range(4) for j in range(4)}
    evals, v_mat = _jacobi4(n_mat, v_mat)

    best = evals[0]
    q4 = [v_mat[(k, 0)] for k in range(4)]
    for j in range(1, 4):
        better = evals[j] > best
        q4 = [jnp.where(better, v_mat[(k, j)], q4[k]) for k in range(4)]
        best = jnp.where(better, evals[j], best)
    qn = f32(1.0) / jnp.sqrt(q4[0] ** 2 + q4[1] ** 2 + q4[2] ** 2 + q4[3] ** 2)
    qw, qx, qy, qz = [c * qn for c in q4]

    r00 = 1 - 2 * (qy * qy + qz * qz)
    r01 = 2 * (qx * qy - qw * qz)
    r02 = 2 * (qx * qz + qw * qy)
    r10 = 2 * (qx * qy + qw * qz)
    r11 = 1 - 2 * (qx * qx + qz * qz)
    r12 = 2 * (qy * qz - qw * qx)
    r20 = 2 * (qx * qz - qw * qy)
    r21 = 2 * (qy * qz + qw * qx)
    r22 = 1 - 2 * (qx * qx + qy * qy)
    rot = [[r00, r01, r02], [r10, r11, r12], [r20, r21, r22]]
    t_vec = [ref_c[i] - (rot[i][0] * src_c[0] + rot[i][1] * src_c[1]
                         + rot[i][2] * src_c[2]) for i in range(3)]

    ri = lax.broadcasted_iota(jnp.int32, (4, 4), 0)
    ci = lax.broadcasted_iota(jnp.int32, (4, 4), 1)
    t_out = jnp.where((ri == 3) & (ci == 3), f32(1.0), f32(0.0))
    for i in range(3):
        for j in range(3):
            t_out = jnp.where((ri == i) & (ci == j), rot[i][j], t_out)
        t_out = jnp.where((ri == i) & (ci == 3), t_vec[i], t_out)
    out_ref[...] = t_out


_tc_main = pl.pallas_call(
    _tc_body,
    out_shape=jax.ShapeDtypeStruct((4, 4), jnp.float32),
    in_specs=[
        pl.BlockSpec(memory_space=pltpu.VMEM),
        pl.BlockSpec(memory_space=pltpu.VMEM),
        pl.BlockSpec(memory_space=pltpu.VMEM),
        pl.BlockSpec(memory_space=pltpu.VMEM),
        pl.BlockSpec(memory_space=pltpu.SMEM),
    ],
    out_specs=pl.BlockSpec(memory_space=pltpu.VMEM),
)


def kernel(ref_knn_masks, src_knn_masks, ref_knn_indices, src_knn_indices,
           score_mat, src_points_f, ref_points_f, distance_threshold):
    f32 = jnp.float32
    planes = _sc_gather()(
        ref_points_f.reshape(-1), src_points_f.reshape(-1),
        ref_knn_indices.astype(jnp.int32),
        src_knn_indices.astype(jnp.int32),
    )
    conf = jnp.reshape(distance_threshold.astype(f32), (1, 1))
    t_out = _tc_main(jnp.transpose(score_mat, (1, 2, 0)),
                     planes.reshape(6, R, B),
                     ref_knn_masks.T, src_knn_masks.T, conf)
    return t_out


# speedup vs baseline: 1.6341x; 1.6341x over previous
import functools

import jax
import jax.numpy as jnp
from jax import lax
from jax.experimental import pallas as pl
from jax.experimental.pallas import tpu as pltpu
from jax.experimental.pallas import tpu_sc as plsc

B, R, S = 256, 64, 64
N_PTS = 20000
K = 3
PAD_D = 16
N_IDX = 2 * B * R
N_WORKERS = 32
IDX_PER_W = N_IDX // N_WORKERS


PER_W = B * R // N_WORKERS


def _sc_gather_body(tbl, refi_hbm, srci_hbm, out_hbm,
                    refi_v, srci_v, idx3_v, vals_v, sem):
    wid = lax.axis_index("s") * 2 + lax.axis_index("c")
    base = wid * PER_W
    cp_r = pltpu.async_copy(refi_hbm.at[pl.ds(base, PER_W)], refi_v, sem)
    cp_s = pltpu.async_copy(srci_hbm.at[pl.ds(base, PER_W)], srci_v, sem)
    cp_r.wait()
    cp_s.wait()
    copies = []
    for cloud, idx_v in enumerate((refi_v, srci_v)):
        for c in range(3):
            k = 3 * cloud + c
            for j in range(PER_W // 16):
                sl = pl.ds(j * 16, 16)
                idx3_v[k][sl] = idx_v[sl] + k * N_PTS
            copies.append(pltpu.async_copy(tbl.at[idx3_v[k]], vals_v[k], sem))
    for k, cp in enumerate(copies):
        cp.wait()
        pltpu.sync_copy(vals_v[k], out_hbm.at[k, pl.ds(base, PER_W)])


@functools.cache
def _sc_gather():
    return pl.kernel(
        _sc_gather_body,
        out_type=jax.ShapeDtypeStruct((6, B * R), jnp.float32),
        mesh=plsc.VectorSubcoreMesh(core_axis_name="c", subcore_axis_name="s"),
        scratch_types=[
            pltpu.VMEM((PER_W,), jnp.int32),
            pltpu.VMEM((PER_W,), jnp.int32),
            [pltpu.VMEM((PER_W,), jnp.int32) for _ in range(6)],
            [pltpu.VMEM((PER_W,), jnp.float32) for _ in range(6)],
            pltpu.SemaphoreType.DMA,
        ],
        compiler_params=pltpu.CompilerParams(use_tc_tiling_on_sc=False,
                                             needs_layout_passes=False),
    )


def _jacobi4(n_mat, v_mat, sweeps=6):
    def get(i, j):
        return n_mat[(i, j)] if i <= j else n_mat[(j, i)]

    def put(i, j, val):
        n_mat[(i, j) if i <= j else (j, i)] = val

    for _ in range(sweeps):
        for p in range(4):
            for q in range(p + 1, 4):
                apq = get(p, q)
                app = get(p, p)
                aqq = get(q, q)
                tau = (aqq - app) / (2.0 * apq + 1e-30)
                t = jnp.sign(tau) / (jnp.abs(tau) + jnp.sqrt(1.0 + tau * tau))
                small = jnp.abs(apq) < 1e-12
                c = jnp.where(small, 1.0, 1.0 / jnp.sqrt(1.0 + t * t))
                s = jnp.where(small, 0.0, t * c)
                for k in range(4):
                    if k != p and k != q:
                        akp = get(k, p)
                        akq = get(k, q)
                        put(k, p, c * akp - s * akq)
                        put(k, q, s * akp + c * akq)
                put(p, p, app - t * apq)
                put(q, q, aqq + t * apq)
                put(p, q, jnp.float32(0.0) * apq)
                for k in range(4):
                    vkp = v_mat[(k, p)]
                    vkq = v_mat[(k, q)]
                    v_mat[(k, p)] = c * vkp - s * vkq
                    v_mat[(k, q)] = s * vkp + c * vkq
    return [n_mat[(i, i)] for i in range(4)], v_mat


def _tc_body(score_ref, planes_ref, maskr_ref, masks_ref, conf_ref, out_ref):
    f32 = jnp.float32
    e = score_ref[...]
    conf = conf_ref[0, 0]
    logconf = jnp.log(conf)
    neg = f32(-jnp.inf)

    m1 = jnp.max(e, axis=1, keepdims=True)
    e1 = jnp.where(e >= m1, neg, e)
    m2 = jnp.max(e1, axis=1, keepdims=True)
    e2 = jnp.where(e1 >= m2, neg, e1)
    m3r = jnp.max(e2, axis=1, keepdims=True)

    c1 = jnp.max(e, axis=0, keepdims=True)
    f1 = jnp.where(e >= c1, neg, e)
    c2 = jnp.max(f1, axis=0, keepdims=True)
    f2 = jnp.where(f1 >= c2, neg, f1)
    m3c = jnp.max(f2, axis=0, keepdims=True)

    maskr = maskr_ref[...].astype(f32)
    masks = masks_ref[...].astype(f32)
    w = jnp.where((e >= m3r) & (e >= m3c) & (e > logconf), f32(1.0), f32(0.0))
    w = w * maskr[:, None, :] * masks[None, :, :]

    wr = jnp.sum(w, axis=1)
    ws = jnp.sum(w, axis=0)
    w_total = jnp.sum(wr)

    refc_p = [planes_ref[i] for i in range(3)]
    srcc_p = [planes_ref[3 + i] for i in range(3)]

    p_s = [jnp.sum(wr * rp) for rp in refc_p]
    q_s = [jnp.sum(ws * sp) for sp in srcc_p]

    g_s = []
    for d in range(3):
        yd = jnp.sum(w * srcc_p[d][None, :, :], axis=1)
        g_s.append([jnp.sum(yd * rp) for rp in refc_p])

    sw = w_total + f32(1e-8)
    ref_c = [p / sw for p in p_s]
    src_c = [q / sw for q in q_s]
    h = [[g_s[d][ee] - src_c[d] * p_s[ee] - q_s[d] * ref_c[ee]
          + w_total * src_c[d] * ref_c[ee] for ee in range(3)] for d in range(3)]

    sxx, sxy, sxz = h[0]
    syx, syy, syz = h[1]
    szx, szy, szz = h[2]
    n_mat = {
        (0, 0): sxx + syy + szz, (0, 1): syz - szy, (0, 2): szx - sxz, (0, 3): sxy - syx,
        (1, 1): sxx - syy - szz, (1, 2): sxy + syx, (1, 3): szx + sxz,
        (2, 2): -sxx + syy - szz, (2, 3): syz + szy,
        (3, 3): -sxx - syy + szz,
    }
    v_mat = {(i, j): f32(1.0) if i == j else f32(0.0)
             for i in range(4) for j in range(4)}
    evals, v_mat = _jacobi4(n_mat, v_mat)

    best = evals[0]
    q4 = [v_mat[(k, 0)] for k in range(4)]
    for j in range(1, 4):
        better = evals[j] > best
        q4 = [jnp.where(better, v_mat[(k, j)], q4[k]) for k in range(4)]
        best = jnp.where(better, evals[j], best)
    qn = f32(1.0) / jnp.sqrt(q4[0] ** 2 + q4[1] ** 2 + q4[2] ** 2 + q4[3] ** 2)
    qw, qx, qy, qz = [c * qn for c in q4]

    r00 = 1 - 2 * (qy * qy + qz * qz)
    r01 = 2 * (qx * qy - qw * qz)
    r02 = 2 * (qx * qz + qw * qy)
    r10 = 2 * (qx * qy + qw * qz)
    r11 = 1 - 2 * (qx * qx + qz * qz)
    r12 = 2 * (qy * qz - qw * qx)
    r20 = 2 * (qx * qz - qw * qy)
    r21 = 2 * (qy * qz + qw * qx)
    r22 = 1 - 2 * (qx * qx + qy * qy)
    rot = [[r00, r01, r02], [r10, r11, r12], [r20, r21, r22]]
    t_vec = [ref_c[i] - (rot[i][0] * src_c[0] + rot[i][1] * src_c[1]
                         + rot[i][2] * src_c[2]) for i in range(3)]

    ri = lax.broadcasted_iota(jnp.int32, (4, 4), 0)
    ci = lax.broadcasted_iota(jnp.int32, (4, 4), 1)
    t_out = jnp.where((ri == 3) & (ci == 3), f32(1.0), f32(0.0))
    for i in range(3):
        for j in range(3):
            t_out = jnp.where((ri == i) & (ci == j), rot[i][j], t_out)
        t_out = jnp.where((ri == i) & (ci == 3), t_vec[i], t_out)
    out_ref[...] = t_out


_tc_main = pl.pallas_call(
    _tc_body,
    out_shape=jax.ShapeDtypeStruct((4, 4), jnp.float32),
    in_specs=[
        pl.BlockSpec(memory_space=pltpu.VMEM),
        pl.BlockSpec(memory_space=pltpu.VMEM),
        pl.BlockSpec(memory_space=pltpu.VMEM),
        pl.BlockSpec(memory_space=pltpu.VMEM),
        pl.BlockSpec(memory_space=pltpu.SMEM),
    ],
    out_specs=pl.BlockSpec(memory_space=pltpu.VMEM),
)


def kernel(ref_knn_masks, src_knn_masks, ref_knn_indices, src_knn_indices,
           score_mat, src_points_f, ref_points_f, distance_threshold):
    f32 = jnp.float32
    tbl = jnp.concatenate([jnp.transpose(ref_points_f).reshape(-1),
                           jnp.transpose(src_points_f).reshape(-1)])
    planes = _sc_gather()(
        tbl,
        ref_knn_indices.T.reshape(-1).astype(jnp.int32),
        src_knn_indices.T.reshape(-1).astype(jnp.int32),
    )
    conf = jnp.reshape(distance_threshold.astype(f32), (1, 1))
    t_out = _tc_main(jnp.transpose(score_mat, (1, 2, 0)),
                     planes.reshape(6, R, B),
                     ref_knn_masks.T, src_knn_masks.T, conf)
    return t_out
